# baseline (speedup 1.0000x reference)
import functools

import jax
import jax.numpy as jnp
from jax import lax
from jax.experimental import pallas as pl
from jax.experimental.pallas import tpu as pltpu
from jax.experimental.pallas import tpu_sc as plsc

D = 64
NC = 2
NS = 16
NW = NC * NS
CHUNK = 128
K = 4
SUPER = CHUNK * K
TB = 4


def _gather_sc(idx3, table2):
    _, n_chunks, _ = idx3.shape
    b_per_w = n_chunks * CHUNK
    n_super = b_per_w // SUPER
    N = NW * b_per_w
    mesh = plsc.VectorSubcoreMesh(core_axis_name="c", subcore_axis_name="s")

    @functools.partial(
        pl.kernel,
        mesh=mesh,
        out_type=jax.ShapeDtypeStruct((N // SUPER, SUPER, 2 * D), jnp.float32),
        compiler_params=pltpu.CompilerParams(use_tc_tiling_on_sc=False),
        scratch_types=[
            pltpu.VMEM((n_chunks, CHUNK), jnp.int32),
            pltpu.VMEM((SUPER, 2 * D), jnp.float32),
            pltpu.SemaphoreType.DMA,
        ],
    )
    def k(idx_hbm, table_hbm, out_hbm, idx_v, rows_v, sem):
        wid = lax.axis_index("s") * NC + lax.axis_index("c")
        sbase = wid * n_super
        pltpu.sync_copy(idx_hbm.at[wid], idx_v)

        def body(s, _):
            handles = []
            for j in range(K):
                handles.append(pltpu.async_copy(
                    table_hbm.at[idx_v.at[s * K + j]],
                    rows_v.at[pl.ds(j * CHUNK, CHUNK)],
                    sem,
                ))
            for h in handles:
                h.wait()
            pltpu.sync_copy(rows_v, out_hbm.at[sbase + s])
            return 0

        lax.fori_loop(0, n_super, body, 0)

    return k(idx3, table2)


def _proj_tc(x3, we3, wo3, b_col, p3, T, B):
    def body(x_ref, we_ref, wo_ref, b_ref, p_ref, o_ref):
        x = x_ref[...]
        dn = (((2,), (2,)), ((0,), (0,)))
        ye = lax.dot_general(we_ref[...], x, dimension_numbers=dn,
                             preferred_element_type=jnp.float32)
        yo = lax.dot_general(wo_ref[...], x, dimension_numbers=dn,
                             preferred_element_type=jnp.float32)
        p = p_ref[...]
        o_ref[...] = ye + p * (yo - ye) + b_ref[...]

    return pl.pallas_call(
        body,
        grid=(T // TB,),
        in_specs=[
            pl.BlockSpec((TB, B, 2 * D), lambda i: (i, 0, 0)),
            pl.BlockSpec((TB, D, 2 * D), lambda i: (0, 0, 0)),
            pl.BlockSpec((TB, D, 2 * D), lambda i: (0, 0, 0)),
            pl.BlockSpec((TB, D, 1), lambda i: (0, 0, 0)),
            pl.BlockSpec((TB, 1, B), lambda i: (i, 0, 0)),
        ],
        out_specs=pl.BlockSpec((TB, D, B), lambda i: (i, 0, 0)),
        out_shape=jax.ShapeDtypeStruct((T, D, B), jnp.float32),
    )(x3, we3, wo3, b_col, p3)


def kernel(input_ids, embed_weight, proj_weight, proj_bias):
    B, T = input_ids.shape
    N = B * T
    V = embed_weight.shape[0]

    table2 = embed_weight.reshape(V // 2, 2 * D)
    ids_t = input_ids.T.astype(jnp.int32)
    idx3 = (ids_t // 2).reshape(NW, N // NW // CHUNK, CHUNK)
    p3 = (ids_t % 2).astype(jnp.float32).reshape(T, 1, B)

    gathered = _gather_sc(idx3, table2)
    x3 = gathered.reshape(T, B, 2 * D)

    we3 = jnp.broadcast_to(
        jnp.pad(proj_weight, ((0, 0), (0, D))).reshape(1, D, 2 * D),
        (TB, D, 2 * D))
    wo3 = jnp.broadcast_to(
        jnp.pad(proj_weight, ((0, 0), (D, 0))).reshape(1, D, 2 * D),
        (TB, D, 2 * D))
    b_col = jnp.broadcast_to(proj_bias.reshape(1, D, 1), (TB, D, 1))

    y3 = _proj_tc(x3, we3, wo3, b_col, p3, T, B)
    return y3.transpose(2, 0, 1)


# speedup vs baseline: 1.0509x; 1.0509x over previous
import functools

import jax
import jax.numpy as jnp
from jax import lax
from jax.experimental import pallas as pl
from jax.experimental.pallas import tpu as pltpu
from jax.experimental.pallas import tpu_sc as plsc

D = 64
NC = 2
NS = 16
NW = NC * NS
CHUNK = 128
K = 4
SUPER = CHUNK * K


def _gather_sc(idx3, table):
    _, n_chunks, _ = idx3.shape
    b_per_w = n_chunks * CHUNK
    n_super = b_per_w // SUPER
    N = NW * b_per_w
    mesh = plsc.VectorSubcoreMesh(core_axis_name="c", subcore_axis_name="s")

    @functools.partial(
        pl.kernel,
        mesh=mesh,
        out_type=jax.ShapeDtypeStruct((N // SUPER, SUPER, D), jnp.float32),
        compiler_params=pltpu.CompilerParams(use_tc_tiling_on_sc=False),
        scratch_types=[
            pltpu.VMEM((n_chunks, CHUNK), jnp.int32),
            pltpu.VMEM((SUPER, D), jnp.float32),
            pltpu.SemaphoreType.DMA,
        ],
    )
    def k(idx_hbm, table_hbm, out_hbm, idx_v, rows_v, sem):
        wid = lax.axis_index("s") * NC + lax.axis_index("c")
        sbase = wid * n_super
        pltpu.sync_copy(idx_hbm.at[wid], idx_v)

        def body(s, _):
            handles = []
            for j in range(K):
                handles.append(pltpu.async_copy(
                    table_hbm.at[idx_v.at[s * K + j]],
                    rows_v.at[pl.ds(j * CHUNK, CHUNK)],
                    sem,
                ))
            for h in handles:
                h.wait()
            pltpu.sync_copy(rows_v, out_hbm.at[sbase + s])
            return 0

        lax.fori_loop(0, n_super, body, 0)

    return k(idx3, table)


def _proj_tc(x3, w3, b_col, T, B):
    TB = 8
    P = B // 2

    def body(x_ref, w_ref, b_ref, o_ref):
        x = x_ref[...]
        for h in range(2):
            xh = x[:, :, h * D:(h + 1) * D]
            yh = lax.dot_general(
                w_ref[...], xh,
                dimension_numbers=(((1,), (2,)), ((0,), (0,))),
                preferred_element_type=jnp.float32,
            )
            o_ref[:, :, h * P:(h + 1) * P] = yh + b_ref[...]

    return pl.pallas_call(
        body,
        grid=(T // TB,),
        in_specs=[
            pl.BlockSpec((TB, P, 2 * D), lambda i: (i, 0, 0)),
            pl.BlockSpec((TB, D, D), lambda i: (0, 0, 0)),
            pl.BlockSpec((TB, D, 1), lambda i: (0, 0, 0)),
        ],
        out_specs=pl.BlockSpec((TB, D, B), lambda i: (i, 0, 0)),
        out_shape=jax.ShapeDtypeStruct((T, D, B), jnp.float32),
    )(x3, w3, b_col)


def kernel(input_ids, embed_weight, proj_weight, proj_bias):
    B, T = input_ids.shape
    N = B * T
    ids_perm = (
        input_ids.T.astype(jnp.int32)
        .reshape(T, 2, B // 2)
        .transpose(0, 2, 1)
        .reshape(NW, N // NW // CHUNK, CHUNK)
    )
    gathered = _gather_sc(ids_perm, embed_weight)
    x3 = gathered.reshape(T, B // 2, 2 * D)
    w3 = jnp.broadcast_to(proj_weight.T.reshape(1, D, D), (8, D, D))
    b_col = jnp.broadcast_to(proj_bias.reshape(1, D, 1), (8, D, 1))
    y3 = _proj_tc(x3, w3, b_col, T, B)
    return y3.transpose(2, 0, 1)


# speedup vs baseline: 1.0681x; 1.0164x over previous
import functools

import jax
import jax.numpy as jnp
from jax import lax
from jax.experimental import pallas as pl
from jax.experimental.pallas import tpu as pltpu
from jax.experimental.pallas import tpu_sc as plsc

D = 64
NC = 2
NS = 16
NW = NC * NS
CHUNK = 128
K = 8
SUPER = CHUNK * K


def _gather_sc(idx3, table):
    _, n_chunks, _ = idx3.shape
    b_per_w = n_chunks * CHUNK
    n_super = b_per_w // SUPER
    N = NW * b_per_w
    mesh = plsc.VectorSubcoreMesh(core_axis_name="c", subcore_axis_name="s")

    @functools.partial(
        pl.kernel,
        mesh=mesh,
        out_type=jax.ShapeDtypeStruct((N // SUPER, SUPER, D), jnp.float32),
        compiler_params=pltpu.CompilerParams(use_tc_tiling_on_sc=False),
        scratch_types=[
            pltpu.VMEM((n_chunks, CHUNK), jnp.int32),
            pltpu.VMEM((SUPER, D), jnp.float32),
            pltpu.SemaphoreType.DMA,
        ],
    )
    def k(idx_hbm, table_hbm, out_hbm, idx_v, rows_v, sem):
        wid = lax.axis_index("s") * NC + lax.axis_index("c")
        sbase = wid * n_super
        pltpu.sync_copy(idx_hbm.at[wid], idx_v)

        def body(s, _):
            handles = []
            for j in range(K):
                handles.append(pltpu.async_copy(
                    table_hbm.at[idx_v.at[s * K + j]],
                    rows_v.at[pl.ds(j * CHUNK, CHUNK)],
                    sem,
                ))
            for h in handles:
                h.wait()
            pltpu.sync_copy(rows_v, out_hbm.at[sbase + s])
            return 0

        lax.fori_loop(0, n_super, body, 0)

    return k(idx3, table)


def _proj_tc(x3, w3, b_col, T, B):
    TB = 8
    P = B // 2

    def body(x_ref, w_ref, b_ref, o_ref):
        x = x_ref[...]
        for h in range(2):
            xh = x[:, :, h * D:(h + 1) * D]
            yh = lax.dot_general(
                w_ref[...], xh,
                dimension_numbers=(((1,), (2,)), ((0,), (0,))),
                preferred_element_type=jnp.float32,
            )
            o_ref[:, :, h * P:(h + 1) * P] = yh + b_ref[...]

    return pl.pallas_call(
        body,
        grid=(T // TB,),
        in_specs=[
            pl.BlockSpec((TB, P, 2 * D), lambda i: (i, 0, 0)),
            pl.BlockSpec((TB, D, D), lambda i: (0, 0, 0)),
            pl.BlockSpec((TB, D, 1), lambda i: (0, 0, 0)),
        ],
        out_specs=pl.BlockSpec((TB, D, B), lambda i: (i, 0, 0)),
        out_shape=jax.ShapeDtypeStruct((T, D, B), jnp.float32),
    )(x3, w3, b_col)


def kernel(input_ids, embed_weight, proj_weight, proj_bias):
    B, T = input_ids.shape
    N = B * T
    ids_perm = (
        input_ids.T.astype(jnp.int32)
        .reshape(T, 2, B // 2)
        .transpose(0, 2, 1)
        .reshape(NW, N // NW // CHUNK, CHUNK)
    )
    gathered = _gather_sc(ids_perm, embed_weight)
    x3 = gathered.reshape(T, B // 2, 2 * D)
    w3 = jnp.broadcast_to(proj_weight.T.reshape(1, D, D), (8, D, D))
    b_col = jnp.broadcast_to(proj_bias.reshape(1, D, 1), (8, D, 1))
    y3 = _proj_tc(x3, w3, b_col, T, B)
    return y3.transpose(2, 0, 1)


# speedup vs baseline: 1.0742x; 1.0057x over previous
import functools

import jax
import jax.numpy as jnp
from jax import lax
from jax.experimental import pallas as pl
from jax.experimental.pallas import tpu as pltpu
from jax.experimental.pallas import tpu_sc as plsc

D = 64
NC = 2
NS = 16
NW = NC * NS
CHUNK = 128
K = 4
SUPER = CHUNK * K


def _gather_sc(idx3, table):
    _, n_chunks, _ = idx3.shape
    b_per_w = n_chunks * CHUNK
    n_super = b_per_w // SUPER
    N = NW * b_per_w
    mesh = plsc.VectorSubcoreMesh(core_axis_name="c", subcore_axis_name="s")

    @functools.partial(
        pl.kernel,
        mesh=mesh,
        out_type=jax.ShapeDtypeStruct((N // SUPER, SUPER, D), jnp.float32),
        compiler_params=pltpu.CompilerParams(use_tc_tiling_on_sc=False),
        scratch_types=[
            pltpu.VMEM((n_chunks, CHUNK), jnp.int32),
            pltpu.VMEM((2, SUPER, D), jnp.float32),
            pltpu.SemaphoreType.DMA,
            pltpu.SemaphoreType.DMA,
        ],
    )
    def k(idx_hbm, table_hbm, out_hbm, idx_v, rows_v, sem, sem_out):
        wid = lax.axis_index("s") * NC + lax.axis_index("c")
        sbase = wid * n_super
        pltpu.sync_copy(idx_hbm.at[wid], idx_v)

        def wait_store(slot):
            pltpu.make_async_copy(
                rows_v.at[slot], out_hbm.at[0], sem_out).wait()

        def body(s, _):
            slot = lax.rem(s, 2)

            @pl.when(s >= 2)
            def _():
                wait_store(slot)

            handles = []
            for j in range(K):
                handles.append(pltpu.async_copy(
                    table_hbm.at[idx_v.at[s * K + j]],
                    rows_v.at[slot, pl.ds(j * CHUNK, CHUNK)],
                    sem,
                ))
            for h in handles:
                h.wait()
            pltpu.async_copy(rows_v.at[slot], out_hbm.at[sbase + s], sem_out)
            return 0

        lax.fori_loop(0, n_super, body, 0)
        wait_store(n_super % 2)
        wait_store((n_super + 1) % 2)

    return k(idx3, table)


def _proj_tc(x3, w3, b_col, T, B):
    TB = 8
    P = B // 2

    def body(x_ref, w_ref, b_ref, o_ref):
        x = x_ref[...]
        for h in range(2):
            xh = x[:, :, h * D:(h + 1) * D]
            yh = lax.dot_general(
                w_ref[...], xh,
                dimension_numbers=(((1,), (2,)), ((0,), (0,))),
                preferred_element_type=jnp.float32,
            )
            o_ref[:, :, h * P:(h + 1) * P] = yh + b_ref[...]

    return pl.pallas_call(
        body,
        grid=(T // TB,),
        in_specs=[
            pl.BlockSpec((TB, P, 2 * D), lambda i: (i, 0, 0)),
            pl.BlockSpec((TB, D, D), lambda i: (0, 0, 0)),
            pl.BlockSpec((TB, D, 1), lambda i: (0, 0, 0)),
        ],
        out_specs=pl.BlockSpec((TB, D, B), lambda i: (i, 0, 0)),
        out_shape=jax.ShapeDtypeStruct((T, D, B), jnp.float32),
    )(x3, w3, b_col)


def kernel(input_ids, embed_weight, proj_weight, proj_bias):
    B, T = input_ids.shape
    N = B * T
    ids_perm = (
        input_ids.T.astype(jnp.int32)
        .reshape(T, 2, B // 2)
        .transpose(0, 2, 1)
        .reshape(NW, N // NW // CHUNK, CHUNK)
    )
    gathered = _gather_sc(ids_perm, embed_weight)
    x3 = gathered.reshape(T, B // 2, 2 * D)
    w3 = jnp.broadcast_to(proj_weight.T.reshape(1, D, D), (8, D, D))
    b_col = jnp.broadcast_to(proj_bias.reshape(1, D, 1), (8, D, 1))
    y3 = _proj_tc(x3, w3, b_col, T, B)
    return y3.transpose(2, 0, 1)


# speedup vs baseline: 1.2782x; 1.1899x over previous
import functools

import jax
import jax.numpy as jnp
from jax import lax
from jax.experimental import pallas as pl
from jax.experimental.pallas import tpu as pltpu
from jax.experimental.pallas import tpu_sc as plsc

D = 64
NC = 2
NS = 16
NW = NC * NS
CHUNK = 128
K = 4


def _permute_idx_sc(ids_t):
    T, B = ids_t.shape
    P64 = 64
    mesh = plsc.VectorSubcoreMesh(core_axis_name="c", subcore_axis_name="s")

    @functools.partial(
        pl.kernel,
        mesh=mesh,
        out_type=jax.ShapeDtypeStruct((NW, T, 2 * P64), jnp.int32),
        compiler_params=pltpu.CompilerParams(
            use_tc_tiling_on_sc=False, needs_layout_passes=False),
        scratch_types=[
            pltpu.VMEM((T, P64), jnp.int32),
            pltpu.VMEM((T, P64), jnp.int32),
            pltpu.VMEM((T, 2 * P64), jnp.int32),
            pltpu.SemaphoreType.DMA,
        ],
    )
    def k(ids_hbm, out_hbm, iva, ivb, ov, sem):
        wid = lax.axis_index("s") * NC + lax.axis_index("c")
        c0 = P64 * wid
        ha = pltpu.async_copy(ids_hbm.at[:, pl.ds(c0, P64)], iva, sem)
        hb = pltpu.async_copy(ids_hbm.at[:, pl.ds(B // 2 + c0, P64)], ivb, sem)
        ha.wait()
        hb.wait()
        lane2 = 2 * jax.lax.broadcasted_iota(jnp.int32, (16,), 0)

        @plsc.parallel_loop(0, T, step=1, unroll=2)
        def row(t):
            tv = jnp.full((16,), t, jnp.int32)
            for k4 in range(P64 // 16):
                ia = lane2 + 32 * k4
                xa = iva[t, pl.ds(16 * k4, 16)]
                plsc.store_scatter(ov, [tv, ia], xa)
                xb = ivb[t, pl.ds(16 * k4, 16)]
                plsc.store_scatter(ov, [tv, ia + 1], xb)

        pltpu.sync_copy(ov, out_hbm.at[wid])

    return k(ids_t)


def _gather_sc(idx3, table):
    _, T, _ = idx3.shape
    mesh = plsc.VectorSubcoreMesh(core_axis_name="c", subcore_axis_name="s")

    @functools.partial(
        pl.kernel,
        mesh=mesh,
        out_type=jax.ShapeDtypeStruct((T * NW, CHUNK, D), jnp.float32),
        compiler_params=pltpu.CompilerParams(use_tc_tiling_on_sc=False),
        scratch_types=[
            pltpu.VMEM((T, CHUNK), jnp.int32),
            pltpu.VMEM((2 * K, CHUNK, D), jnp.float32),
            pltpu.SemaphoreType.DMA,
            pltpu.SemaphoreType.DMA,
        ],
    )
    def k(idx_hbm, table_hbm, out_hbm, idx_v, rows_v, sem, sem_out):
        wid = lax.axis_index("s") * NC + lax.axis_index("c")
        pltpu.sync_copy(idx_hbm.at[wid], idx_v)

        def start_gather(c, slot):
            pltpu.async_copy(
                table_hbm.at[idx_v.at[c]], rows_v.at[slot], sem)

        def wait_gather(slot):
            pltpu.make_async_copy(
                table_hbm.at[idx_v.at[0]], rows_v.at[slot], sem).wait()

        def wait_store(slot):
            pltpu.make_async_copy(
                rows_v.at[slot], out_hbm.at[0], sem_out).wait()

        for c in range(K):
            start_gather(c, c)

        def body(c, _):
            slot = lax.rem(c, 2 * K)
            wait_gather(slot)

            @pl.when(c + K < T)
            def _():
                nslot = lax.rem(c + K, 2 * K)

                @pl.when(c >= K)
                def _():
                    wait_store(nslot)

                start_gather(c + K, nslot)

            pltpu.async_copy(
                rows_v.at[slot], out_hbm.at[c * NW + wid], sem_out)
            return 0

        lax.fori_loop(0, T, body, 0)
        for c in range(T - 2 * K, T):
            wait_store(c % (2 * K))

    return k(idx3, table)


def _proj_tc(x3, w3, b_col, T, B):
    TB = 8
    P = B // 2

    def body(x_ref, w_ref, b_ref, o_ref):
        x = x_ref[...]
        for h in range(2):
            xh = x[:, :, h * D:(h + 1) * D]
            yh = lax.dot_general(
                w_ref[...], xh,
                dimension_numbers=(((1,), (2,)), ((0,), (0,))),
                preferred_element_type=jnp.float32,
            )
            o_ref[:, :, h * P:(h + 1) * P] = yh + b_ref[...]

    return pl.pallas_call(
        body,
        grid=(T // TB,),
        in_specs=[
            pl.BlockSpec((TB, P, 2 * D), lambda i: (i, 0, 0)),
            pl.BlockSpec((TB, D, D), lambda i: (0, 0, 0)),
            pl.BlockSpec((TB, D, 1), lambda i: (0, 0, 0)),
        ],
        out_specs=pl.BlockSpec((TB, D, B), lambda i: (i, 0, 0)),
        out_shape=jax.ShapeDtypeStruct((T, D, B), jnp.float32),
    )(x3, w3, b_col)


def kernel(input_ids, embed_weight, proj_weight, proj_bias):
    B, T = input_ids.shape
    ids_t = input_ids.T.astype(jnp.int32)
    idx3 = _permute_idx_sc(ids_t)
    gathered = _gather_sc(idx3, embed_weight)
    x3 = gathered.reshape(T, B // 2, 2 * D)
    w3 = jnp.broadcast_to(proj_weight.T.reshape(1, D, D), (8, D, D))
    b_col = jnp.broadcast_to(proj_bias.reshape(1, D, 1), (8, D, 1))
    y3 = _proj_tc(x3, w3, b_col, T, B)
    return y3.transpose(2, 0, 1)
